# baseline (speedup 1.0000x reference)
import functools

import jax
import jax.numpy as jnp
from jax import lax
from jax.experimental import pallas as pl
from jax.experimental.pallas import tpu as pltpu
from jax.experimental.pallas import tpu_sc as plsc

NC = 2
NS = 16
NW = NC * NS

V = 1000000
D = 32
B = 4096 * 200
BPW = B // NW

VB = 128
NBLK = V // VB
VTAIL = V - NBLK * VB
BLK_PER_W = (NBLK + NW - 1) // NW
LINES = V // 4

_mesh = plsc.VectorSubcoreMesh(core_axis_name="c", subcore_axis_name="s")


@functools.partial(
    pl.kernel,
    mesh=_mesh,
    compiler_params=pltpu.CompilerParams(
        use_tc_tiling_on_sc=True, needs_layout_passes=False
    ),
    out_type=jax.ShapeDtypeStruct((LINES, 128), jnp.float32),
    scratch_types=[
        pltpu.VMEM((2, 4, 8, VB), jnp.float32),
        pltpu.VMEM((2, 4, 8, VB), jnp.float32),
        pltpu.VMEM((4, 8, VTAIL), jnp.float32),
        pltpu.SemaphoreType.DMA,
        pltpu.SemaphoreType.DMA,
        pltpu.SemaphoreType.DMA,
        pltpu.SemaphoreType.DMA,
    ],
)
def _sc_transpose(tT, tail_T, scratch, buf, bufT, tailbuf, ls0, ls1, ws0, ws1):
    wid = lax.axis_index("s") * NC + lax.axis_index("c")
    lsems = (ls0, ls1)
    wsems = (ws0, ws1)
    lanes = lax.iota(jnp.int32, 16)
    t0 = lanes // 8
    r0 = lanes % 8
    t1 = t0 + 2

    def bid(j):
        return jnp.minimum(wid + j * NW, NBLK - 1)

    def start_load(j, s):
        v0 = pl.multiple_of(bid(j) * VB, VB)
        for q in range(4):
            pltpu.async_copy(
                tT.at[pl.ds(q * 8, 8), pl.ds(v0, VB)], buf.at[s, q], lsems[s]
            )

    def wait_load(s):
        for q in range(4):
            pltpu.make_async_copy(
                tT.at[pl.ds(0, 8), pl.ds(0, VB)], buf.at[s, q], lsems[s]
            ).wait()

    def transpose_into(src, dst, nq):
        @pl.loop(0, nq)
        def _(q):
            for rr in range(8):
                for g4 in range(4):
                    c = jnp.broadcast_to((q * 8 + rr) * 4 + g4, (16,)).astype(
                        jnp.int32
                    )
                    dst[q, rr, pl.ds(g4 * 32, 16)] = plsc.load_gather(
                        src, [t0, r0, c]
                    )
                    dst[q, rr, pl.ds(g4 * 32 + 16, 16)] = plsc.load_gather(
                        src, [t1, r0, c]
                    )

    def start_write(j, s):
        line0 = pl.multiple_of(bid(j) * (VB // 4), 8)
        for q in range(4):
            pltpu.async_copy(
                bufT.at[s, q], scratch.at[pl.ds(line0 + q * 8, 8)], wsems[s]
            )

    def wait_write(s):
        for q in range(4):
            pltpu.make_async_copy(
                bufT.at[s, q], scratch.at[pl.ds(0, 8)], wsems[s]
            ).wait()

    def process(j, s, first):
        wait_load(s)
        if not first:
            wait_write(s)
        transpose_into(buf.at[s], bufT.at[s], 4)
        start_write(j, s)

    start_load(0, 0)

    @pl.loop(1, BLK_PER_W - 1, step=2)
    def _(g):
        start_load(g, 1)
        is_first0 = g == 1

        @pl.when(is_first0)
        def _():
            wait_load(0)
            transpose_into(buf.at[0], bufT.at[0], 4)
            start_write(g - 1, 0)

        @pl.when(jnp.logical_not(is_first0))
        def _():
            process(g - 1, 0, False)

        start_load(g + 1, 0)
        is_first1 = g == 1

        @pl.when(is_first1)
        def _():
            wait_load(1)
            transpose_into(buf.at[1], bufT.at[1], 4)
            start_write(g, 1)

        @pl.when(jnp.logical_not(is_first1))
        def _():
            process(g, 1, False)

    process(BLK_PER_W - 1, 0, False)
    wait_write(1)
    wait_write(0)

    @pl.when(wid == NW - 1)
    def _():
        for q in range(4):
            pltpu.sync_copy(tail_T.at[pl.ds(q * 8, 8)], tailbuf.at[q])
        transpose_into(tailbuf, bufT.at[0], 2)
        for q in range(2):
            pltpu.sync_copy(
                bufT.at[0, q],
                scratch.at[pl.ds(NBLK * (VB // 4) + q * 8, 8)],
            )


G = 128
K = 8
CHUNK = K * G
NCHUNKS = BPW // CHUNK
IDX_ROWS_PER_W = BPW // G


@functools.partial(
    pl.kernel,
    mesh=_mesh,
    compiler_params=pltpu.CompilerParams(use_tc_tiling_on_sc=False),
    out_type=jax.ShapeDtypeStruct((B, D), jnp.float32),
    scratch_types=[
        pltpu.VMEM((2, K, G), jnp.int32),
        pltpu.VMEM((2, CHUNK, D), jnp.float32),
        pltpu.SemaphoreType.DMA,
        pltpu.SemaphoreType.DMA,
    ],
)
def _sc_gather(idx_hbm, table_hbm, out_hbm, idx_v, rows_v, gsem0, gsem1):
    wid = lax.axis_index("s") * NC + lax.axis_index("c")
    idx_row0 = wid * IDX_ROWS_PER_W
    out_row0 = wid * BPW
    gsems = (gsem0, gsem1)

    def load_idx(j, s):
        pltpu.sync_copy(idx_hbm.at[pl.ds(idx_row0 + j * K, K)], idx_v.at[s])

    def start_gather(s):
        for r in range(K):
            pltpu.async_copy(
                table_hbm.at[idx_v.at[s, r]],
                rows_v.at[s, pl.ds(r * G, G)],
                gsems[s],
            )

    def wait_gather(s):
        pltpu.make_async_copy(
            out_hbm.at[pl.ds(0, CHUNK)], rows_v.at[s], gsems[s]
        ).wait()

    def write_out(j, s):
        pltpu.sync_copy(rows_v.at[s], out_hbm.at[pl.ds(out_row0 + j * CHUNK, CHUNK)])

    load_idx(0, 0)
    start_gather(0)

    @pl.loop(1, NCHUNKS - 1, step=2)
    def _(g):
        load_idx(g, 1)
        start_gather(1)
        wait_gather(0)
        write_out(g - 1, 0)
        load_idx(g + 1, 0)
        start_gather(0)
        wait_gather(1)
        write_out(g, 1)

    wait_gather(0)
    write_out(NCHUNKS - 1, 0)


def kernel(words, word_table):
    tT = word_table.T
    tail_T = word_table[NBLK * VB :, :].T
    scratch = _sc_transpose(tT, tail_T)
    flat_idx = words.reshape(B // G, G)
    out = _sc_gather(flat_idx, scratch.reshape(V, D))
    return out.reshape(*words.shape, D)


# speedup vs baseline: 1.1068x; 1.1068x over previous
import functools

import jax
import jax.numpy as jnp
from jax import lax
from jax.experimental import pallas as pl
from jax.experimental.pallas import tpu as pltpu
from jax.experimental.pallas import tpu_sc as plsc

NC = 2
NS = 16
NW = NC * NS

V = 1000000
D = 32
B = 4096 * 200
BPW = B // NW

VB = 128
NBLK = V // VB
VTAIL = V - NBLK * VB
BLK_PER_W = (NBLK + NW - 1) // NW
LINES = V // 4

_mesh = plsc.VectorSubcoreMesh(core_axis_name="c", subcore_axis_name="s")


@functools.partial(
    pl.kernel,
    mesh=_mesh,
    compiler_params=pltpu.CompilerParams(
        use_tc_tiling_on_sc=True, needs_layout_passes=False
    ),
    out_type=jax.ShapeDtypeStruct((LINES, 128), jnp.float32),
    scratch_types=[
        pltpu.VMEM((2, 4, 8, VB), jnp.float32),
        pltpu.VMEM((2, 4, 8, VB), jnp.float32),
        pltpu.VMEM((4, 8, VTAIL), jnp.float32),
        pltpu.SemaphoreType.DMA,
        pltpu.SemaphoreType.DMA,
        pltpu.SemaphoreType.DMA,
        pltpu.SemaphoreType.DMA,
    ],
)
def _sc_transpose(tT, tail_T, scratch, buf, bufT, tailbuf, ls0, ls1, ws0, ws1):
    wid = lax.axis_index("s") * NC + lax.axis_index("c")
    lsems = (ls0, ls1)
    wsems = (ws0, ws1)
    lanes = lax.iota(jnp.int32, 16)
    t0 = lanes // 8
    r0 = lanes % 8
    t1 = t0 + 2

    def bid(j):
        return jnp.minimum(wid + j * NW, NBLK - 1)

    def start_load(j, s):
        v0 = pl.multiple_of(bid(j) * VB, VB)
        for q in range(4):
            pltpu.async_copy(
                tT.at[pl.ds(q * 8, 8), pl.ds(v0, VB)], buf.at[s, q], lsems[s]
            )

    def wait_load(s):
        for q in range(4):
            pltpu.make_async_copy(
                tT.at[pl.ds(0, 8), pl.ds(0, VB)], buf.at[s, q], lsems[s]
            ).wait()

    def transpose_into(src, dst, nq):
        @plsc.parallel_loop(0, nq, unroll=2)
        def _(q):
            for rr in range(8):
                for g4 in range(4):
                    c = jnp.broadcast_to((q * 8 + rr) * 4 + g4, (16,)).astype(
                        jnp.int32
                    )
                    dst[q, rr, pl.ds(g4 * 32, 16)] = plsc.load_gather(
                        src, [t0, r0, c]
                    )
                    dst[q, rr, pl.ds(g4 * 32 + 16, 16)] = plsc.load_gather(
                        src, [t1, r0, c]
                    )

    def start_write(j, s):
        line0 = pl.multiple_of(bid(j) * (VB // 4), 8)
        for q in range(4):
            pltpu.async_copy(
                bufT.at[s, q], scratch.at[pl.ds(line0 + q * 8, 8)], wsems[s]
            )

    def wait_write(s):
        for q in range(4):
            pltpu.make_async_copy(
                bufT.at[s, q], scratch.at[pl.ds(0, 8)], wsems[s]
            ).wait()

    def process(j, s, first):
        wait_load(s)
        if not first:
            wait_write(s)
        transpose_into(buf.at[s], bufT.at[s], 4)
        start_write(j, s)

    start_load(0, 0)

    @pl.loop(1, BLK_PER_W - 1, step=2)
    def _(g):
        start_load(g, 1)
        is_first0 = g == 1

        @pl.when(is_first0)
        def _():
            wait_load(0)
            transpose_into(buf.at[0], bufT.at[0], 4)
            start_write(g - 1, 0)

        @pl.when(jnp.logical_not(is_first0))
        def _():
            process(g - 1, 0, False)

        start_load(g + 1, 0)
        is_first1 = g == 1

        @pl.when(is_first1)
        def _():
            wait_load(1)
            transpose_into(buf.at[1], bufT.at[1], 4)
            start_write(g, 1)

        @pl.when(jnp.logical_not(is_first1))
        def _():
            process(g, 1, False)

    process(BLK_PER_W - 1, 0, False)
    wait_write(1)
    wait_write(0)

    @pl.when(wid == NW - 1)
    def _():
        for q in range(4):
            pltpu.sync_copy(tail_T.at[pl.ds(q * 8, 8)], tailbuf.at[q])
        transpose_into(tailbuf, bufT.at[0], 2)
        for q in range(2):
            pltpu.sync_copy(
                bufT.at[0, q],
                scratch.at[pl.ds(NBLK * (VB // 4) + q * 8, 8)],
            )


G = 128
K = 8
CHUNK = K * G
NCHUNKS = BPW // CHUNK
IDX_ROWS_PER_W = BPW // G


@functools.partial(
    pl.kernel,
    mesh=_mesh,
    compiler_params=pltpu.CompilerParams(use_tc_tiling_on_sc=False),
    out_type=jax.ShapeDtypeStruct((B, D), jnp.float32),
    scratch_types=[
        pltpu.VMEM((2, K, G), jnp.int32),
        pltpu.VMEM((2, CHUNK, D), jnp.float32),
        pltpu.SemaphoreType.DMA,
        pltpu.SemaphoreType.DMA,
    ],
)
def _sc_gather(idx_hbm, table_hbm, out_hbm, idx_v, rows_v, gsem0, gsem1):
    wid = lax.axis_index("s") * NC + lax.axis_index("c")
    idx_row0 = wid * IDX_ROWS_PER_W
    out_row0 = wid * BPW
    gsems = (gsem0, gsem1)

    def load_idx(j, s):
        pltpu.sync_copy(idx_hbm.at[pl.ds(idx_row0 + j * K, K)], idx_v.at[s])

    def start_gather(s):
        for r in range(K):
            pltpu.async_copy(
                table_hbm.at[idx_v.at[s, r]],
                rows_v.at[s, pl.ds(r * G, G)],
                gsems[s],
            )

    def wait_gather(s):
        pltpu.make_async_copy(
            out_hbm.at[pl.ds(0, CHUNK)], rows_v.at[s], gsems[s]
        ).wait()

    def write_out(j, s):
        pltpu.sync_copy(rows_v.at[s], out_hbm.at[pl.ds(out_row0 + j * CHUNK, CHUNK)])

    load_idx(0, 0)
    start_gather(0)

    @pl.loop(1, NCHUNKS - 1, step=2)
    def _(g):
        load_idx(g, 1)
        start_gather(1)
        wait_gather(0)
        write_out(g - 1, 0)
        load_idx(g + 1, 0)
        start_gather(0)
        wait_gather(1)
        write_out(g, 1)

    wait_gather(0)
    write_out(NCHUNKS - 1, 0)


def kernel(words, word_table):
    tT = word_table.T
    tail_T = word_table[NBLK * VB :, :].T
    scratch = _sc_transpose(tT, tail_T)
    flat_idx = words.reshape(B // G, G)
    out = _sc_gather(flat_idx, scratch.reshape(V, D))
    return out.reshape(*words.shape, D)


# speedup vs baseline: 1.3054x; 1.1794x over previous
import functools

import jax
import jax.numpy as jnp
from jax import lax
from jax.experimental import pallas as pl
from jax.experimental.pallas import tpu as pltpu
from jax.experimental.pallas import tpu_sc as plsc

NC = 2
NS = 16
NW = NC * NS

B = 4096 * 200
D = 32
BPW = B // NW

G = 128
K = 8
CHUNK = K * G
NCHUNKS = BPW // CHUNK
IDX_ROWS_PER_W = BPW // G

_mesh = plsc.VectorSubcoreMesh(core_axis_name="c", subcore_axis_name="s")


@functools.partial(
    pl.kernel,
    mesh=_mesh,
    compiler_params=pltpu.CompilerParams(use_tc_tiling_on_sc=False),
    out_type=jax.ShapeDtypeStruct((B, D), jnp.float32),
    scratch_types=[
        pltpu.VMEM((2, K, G), jnp.int32),
        pltpu.VMEM((2, CHUNK, D), jnp.float32),
        pltpu.SemaphoreType.DMA,
        pltpu.SemaphoreType.DMA,
    ],
)
def _sc_gather(idx_hbm, table_hbm, out_hbm, idx_v, rows_v, gsem0, gsem1):
    wid = lax.axis_index("s") * NC + lax.axis_index("c")
    idx_row0 = wid * IDX_ROWS_PER_W
    out_row0 = wid * BPW
    gsems = (gsem0, gsem1)

    def load_idx(j, s):
        pltpu.sync_copy(idx_hbm.at[pl.ds(idx_row0 + j * K, K)], idx_v.at[s])

    def start_gather(s):
        for r in range(K):
            pltpu.async_copy(
                table_hbm.at[idx_v.at[s, r]],
                rows_v.at[s, pl.ds(r * G, G)],
                gsems[s],
            )

    def wait_gather(s):
        pltpu.make_async_copy(
            out_hbm.at[pl.ds(0, CHUNK)], rows_v.at[s], gsems[s]
        ).wait()

    def write_out(j, s):
        pltpu.sync_copy(rows_v.at[s], out_hbm.at[pl.ds(out_row0 + j * CHUNK, CHUNK)])

    load_idx(0, 0)
    start_gather(0)

    @pl.loop(1, NCHUNKS - 1, step=2)
    def _(g):
        load_idx(g, 1)
        start_gather(1)
        wait_gather(0)
        write_out(g - 1, 0)
        load_idx(g + 1, 0)
        start_gather(0)
        wait_gather(1)
        write_out(g, 1)

    wait_gather(0)
    write_out(NCHUNKS - 1, 0)


def kernel(words, word_table):
    flat_idx = words.reshape(B // G, G)
    out = _sc_gather(flat_idx, word_table)
    return out.reshape(*words.shape, D)


# speedup vs baseline: 1.3194x; 1.0108x over previous
import functools

import jax
import jax.numpy as jnp
from jax import lax
from jax.experimental import pallas as pl
from jax.experimental.pallas import tpu as pltpu
from jax.experimental.pallas import tpu_sc as plsc

NC = 2
NS = 16
NW = NC * NS

B = 4096 * 200
D = 32
BPW = B // NW

G = 128
K = 8
CHUNK = K * G
NCHUNKS = BPW // CHUNK
IDX_ROWS_PER_W = BPW // G

_mesh = plsc.VectorSubcoreMesh(core_axis_name="c", subcore_axis_name="s")


@functools.partial(
    pl.kernel,
    mesh=_mesh,
    compiler_params=pltpu.CompilerParams(use_tc_tiling_on_sc=False),
    out_type=jax.ShapeDtypeStruct((B, D), jnp.float32),
    scratch_types=[
        pltpu.VMEM((3, K, G), jnp.int32),
        pltpu.VMEM((3, CHUNK, D), jnp.float32),
        pltpu.SemaphoreType.DMA,
        pltpu.SemaphoreType.DMA,
        pltpu.SemaphoreType.DMA,
        pltpu.SemaphoreType.DMA,
        pltpu.SemaphoreType.DMA,
        pltpu.SemaphoreType.DMA,
    ],
)
def _sc_gather(
    idx_hbm, table_hbm, out_hbm, idx_v, rows_v, g0, g1, g2, w0, w1, w2
):
    wid = lax.axis_index("s") * NC + lax.axis_index("c")
    idx_row0 = wid * IDX_ROWS_PER_W
    out_row0 = wid * BPW
    gsems = (g0, g1, g2)
    wsems = (w0, w1, w2)

    def load_idx(j, s):
        pltpu.sync_copy(idx_hbm.at[pl.ds(idx_row0 + j * K, K)], idx_v.at[s])

    def start_gather(s):
        for r in range(K):
            pltpu.async_copy(
                table_hbm.at[idx_v.at[s, r]],
                rows_v.at[s, pl.ds(r * G, G)],
                gsems[s],
            )

    def wait_gather(s):
        pltpu.make_async_copy(
            out_hbm.at[pl.ds(0, CHUNK)], rows_v.at[s], gsems[s]
        ).wait()

    def start_write(j, s):
        pltpu.async_copy(
            rows_v.at[s], out_hbm.at[pl.ds(out_row0 + j * CHUNK, CHUNK)], wsems[s]
        )

    def wait_write(s):
        pltpu.make_async_copy(
            rows_v.at[s], out_hbm.at[pl.ds(0, CHUNK)], wsems[s]
        ).wait()

    load_idx(0, 0)
    start_gather(0)

    load_idx(1, 1)
    start_gather(1)
    wait_gather(0)
    start_write(0, 0)

    load_idx(2, 2)
    start_gather(2)
    wait_gather(1)
    start_write(1, 1)

    @pl.loop(0, 7)
    def _(t):
        j = 2 + 3 * t
        wait_write(0)
        load_idx(j + 1, 0)
        start_gather(0)
        wait_gather(2)
        start_write(j, 2)
        wait_write(1)
        load_idx(j + 2, 1)
        start_gather(1)
        wait_gather(0)
        start_write(j + 1, 0)
        wait_write(2)
        load_idx(j + 3, 2)
        start_gather(2)
        wait_gather(1)
        start_write(j + 2, 1)

    wait_write(0)
    load_idx(24, 0)
    start_gather(0)
    wait_gather(2)
    start_write(23, 2)

    wait_gather(0)
    start_write(24, 0)

    wait_write(1)
    wait_write(2)
    wait_write(0)


def kernel(words, word_table):
    flat_idx = words.reshape(B // G, G)
    out = _sc_gather(flat_idx, word_table)
    return out.reshape(*words.shape, D)
